# baseline (speedup 1.0000x reference)
import functools

import jax
import jax.numpy as jnp
from jax import lax
from jax.experimental import pallas as pl
from jax.experimental.pallas import tpu as pltpu
from jax.experimental.pallas import tpu_sc as plsc

_B, _N, _OV, _O = 16, 4096, 128, 64
_NW = 32
_RW = _B * _N // _NW
_C = 256


def _gs_body(global_state_ref, W_ref, gs_ref):
    gs_ref[...] = jnp.dot(global_state_ref[...], W_ref[...],
                          preferred_element_type=jnp.float32)


def _sc_body(V_hbm, gs_hbm, gsz_hbm, out_hbm, gsz_v, gs_row_v, row_v):
    cid = lax.axis_index("c")
    sid = lax.axis_index("s")
    wid = sid * 2 + cid
    bidx = wid // 2
    r0 = (wid % 2) * _RW

    pltpu.sync_copy(gsz_hbm, gsz_v)
    pltpu.sync_copy(gs_hbm.at[bidx], gs_row_v)
    gvec = gsz_v[...]
    gsize = gvec[0]
    for k in range(1, _B):
        gsize = jnp.where(bidx == k, gvec[k], gsize)

    gv = [gs_row_v[pl.ds(j * 16, 16)] for j in range(_O // 16)]

    for chunk in range(_RW // _C):
        base = r0 + chunk * _C
        pltpu.sync_copy(V_hbm.at[bidx, pl.ds(base, _C)],
                        row_v.at[:, pl.ds(0, _OV)])

        def fill_row(i, _):
            m = jnp.where(base + i < gsize, 1.0, 0.0)
            for j in range(_O // 16):
                row_v[i, pl.ds(_OV + j * 16, 16)] = gv[j] * m
            return 0

        lax.fori_loop(0, _C, fill_row, 0)
        pltpu.sync_copy(row_v, out_hbm.at[bidx, pl.ds(base, _C)])


@jax.jit
def kernel(V, global_state, graph_size, W):
    b, N, Ov = V.shape
    O = W.shape[1]
    gs = pl.pallas_call(
        _gs_body,
        out_shape=jax.ShapeDtypeStruct((b, O), jnp.float32),
    )(global_state, W)

    sc_assemble = pl.kernel(
        _sc_body,
        out_type=jax.ShapeDtypeStruct((b, N, Ov + O), jnp.float32),
        mesh=plsc.VectorSubcoreMesh(core_axis_name="c", subcore_axis_name="s"),
        scratch_types=[
            pltpu.VMEM((b,), jnp.int32),
            pltpu.VMEM((O,), jnp.float32),
            pltpu.VMEM((_C, Ov + O), jnp.float32),
        ],
    )
    out = sc_assemble(V, gs, graph_size)
    return out, gs
